# baseline (speedup 1.0000x reference)
import functools

import jax
import jax.numpy as jnp
from jax import lax
from jax.experimental import pallas as pl
from jax.experimental.pallas import tpu as pltpu
from jax.experimental.pallas import tpu_sc as plsc

N = 320000
D = 128
H = 64
S = 10000
C = 3200
G = N // C
W = 128
NTILES = 32
RPT = 328
ACC_R = 16 * RPT
OFF1 = 4992
OVR = S + W


def _tc_partials(x_ref, bi_ref, w_ref, v_ref, pout_ref, ovfn_ref, den_ref):
    c = pl.program_id(0)

    @pl.when(c == 0)
    def _init():
        ovfn_ref[...] = jnp.zeros_like(ovfn_ref)
        den_ref[...] = jnp.zeros_like(den_ref)

    x = x_ref[...]
    h = jnp.tanh(lax.dot(x, w_ref[...]))
    beta = lax.dot(h, v_ref[...])
    e = jnp.exp(beta)
    wgt = (x * e).astype(jnp.bfloat16)
    e_bf = e.astype(jnp.bfloat16)

    ids = bi_ref[0]
    base = (jnp.min(ids) // 8) * 8
    local = ids - base
    nwin = jnp.max(local) // W + 1

    row = lax.broadcasted_iota(jnp.int32, (W, C), 0)

    oht0 = (row == local).astype(jnp.bfloat16)
    win_num = lax.dot(oht0, wgt, preferred_element_type=jnp.float32)
    win_den = lax.dot(oht0, e_bf, preferred_element_type=jnp.float32)
    pout_ref[0] = win_num
    den_ref[pl.ds(base, W), :] += win_den

    @pl.when(nwin > 1)
    def _extra():
        def body(k, carry):
            oht = (row + k * W == local).astype(jnp.bfloat16)
            wn = lax.dot(oht, wgt, preferred_element_type=jnp.float32)
            wd = lax.dot(oht, e_bf, preferred_element_type=jnp.float32)
            b = base + k * W
            ovfn_ref[pl.ds(b, W), :] += wn
            den_ref[pl.ds(b, W), :] += wd
            return carry
        lax.fori_loop(1, nwin, body, 0)


@functools.partial(
    pl.kernel,
    out_type=[jax.ShapeDtypeStruct((ACC_R, D), jnp.float32),
              jax.ShapeDtypeStruct((ACC_R, D), jnp.float32)],
    mesh=plsc.VectorSubcoreMesh(core_axis_name="c", subcore_axis_name="s"),
    scratch_types=[pltpu.VMEM((RPT, D), jnp.float32),
                   pltpu.VMEM((W,), jnp.int32),
                   pltpu.VMEM((128,), jnp.int32),
                   pltpu.VMEM_SHARED((ACC_R, D), jnp.float32)],
)
def _sc_merge(pout_hbm, bases_hbm, out0, out1, zbuf, idxv, bsm, shacc):
    cid = lax.axis_index("c")
    sid = lax.axis_index("s")

    pltpu.sync_copy(bases_hbm, bsm)

    @pl.loop(0, RPT)
    def _z(r):
        for jj in range(D // 16):
            zbuf[r, pl.ds(jj * 16, 16)] = jnp.zeros((16,), jnp.float32)
    pltpu.sync_copy(zbuf, shacc.at[pl.ds(sid * RPT, RPT)])
    plsc.subcore_barrier()

    for j in range((G + 15) // 16):
        q = sid + 16 * j

        @pl.when(q < G)
        def _do(q=q):
            b = bsm[pl.ds(q, 16)][0]
            mine = jnp.where(cid == 0, (b < OFF1).astype(jnp.int32),
                             (b >= OFF1).astype(jnp.int32))

            @pl.when(mine == 1)
            def _go():
                pltpu.sync_copy(pout_hbm.at[q], zbuf.at[pl.ds(0, W)])
                blocal = b - cid * OFF1
                for jj in range(W // 16):
                    idxv[pl.ds(jj * 16, 16)] = (
                        lax.broadcasted_iota(jnp.int32, (16,), 0)
                        + (blocal + jj * 16))
                pltpu.sync_copy(zbuf.at[pl.ds(0, W)], shacc.at[idxv],
                                add=True)

    plsc.subcore_barrier()

    @pl.when(cid == 0)
    def _w0():
        pltpu.sync_copy(shacc.at[pl.ds(sid * RPT, RPT)],
                        out0.at[pl.ds(sid * RPT, RPT)])

    @pl.when(cid == 1)
    def _w1():
        pltpu.sync_copy(shacc.at[pl.ds(sid * RPT, RPT)],
                        out1.at[pl.ds(sid * RPT, RPT)])


def _tc_finalize(a0_ref, a1_ref, ovfn_ref, den_ref, out_ref):
    a0 = a0_ref[...]
    a1 = a1_ref[...]
    top = a0[:OFF1]
    mid = a0[OFF1:] + a1[:ACC_R - OFF1]
    bot = a1[ACC_R - OFF1:S - OFF1]
    num = jnp.concatenate([top, mid, bot], axis=0) + ovfn_ref[...]
    den = den_ref[...]
    out_ref[...] = jnp.where(den > 0, num / den, 0.0)


def kernel(embed, batch_index, W_a, V_a):
    bi3 = batch_index.reshape(G, 1, C)
    bases = (batch_index[::C] // 8) * 8
    bases = jnp.concatenate(
        [bases, jnp.zeros((128 - G,), jnp.int32)])

    pout, ovfn, den = pl.pallas_call(
        _tc_partials,
        grid=(G,),
        in_specs=[
            pl.BlockSpec((C, D), lambda c: (c, 0)),
            pl.BlockSpec((1, 1, C), lambda c: (c, 0, 0)),
            pl.BlockSpec((D, H), lambda c: (0, 0)),
            pl.BlockSpec((H, 1), lambda c: (0, 0)),
        ],
        out_specs=[
            pl.BlockSpec((1, W, D), lambda c: (c, 0, 0)),
            pl.BlockSpec((OVR, D), lambda c: (0, 0)),
            pl.BlockSpec((OVR, 1), lambda c: (0, 0)),
        ],
        out_shape=[
            jax.ShapeDtypeStruct((G, W, D), jnp.float32),
            jax.ShapeDtypeStruct((OVR, D), jnp.float32),
            jax.ShapeDtypeStruct((OVR, 1), jnp.float32),
        ],
    )(embed, bi3, W_a, V_a)

    acc0, acc1 = _sc_merge(pout, bases)

    return pl.pallas_call(
        _tc_finalize,
        grid=(1,),
        in_specs=[
            pl.BlockSpec((ACC_R, D), lambda i: (0, 0)),
            pl.BlockSpec((ACC_R, D), lambda i: (0, 0)),
            pl.BlockSpec((S, D), lambda i: (0, 0)),
            pl.BlockSpec((S, 1), lambda i: (0, 0)),
        ],
        out_specs=pl.BlockSpec((S, D), lambda i: (0, 0)),
        out_shape=jax.ShapeDtypeStruct((S, D), jnp.float32),
    )(acc0, acc1, ovfn, den)


# speedup vs baseline: 1.3130x; 1.3130x over previous
import jax
import jax.numpy as jnp
from jax.experimental import pallas as pl
from jax.experimental.pallas import tpu as pltpu

N = 320000
D = 128
H = 64
S = 10000
C = 3200
G = N // C
W = 128


def _attn_kernel(x_ref, bi_ref, w_ref, v_ref, out_ref, acc_ref, den_ref):
    c = pl.program_id(0)

    @pl.when(c == 0)
    def _init():
        acc_ref[...] = jnp.zeros_like(acc_ref)
        den_ref[...] = jnp.zeros_like(den_ref)

    x = x_ref[...]
    h = jnp.tanh(jax.lax.dot(x, w_ref[...]))
    beta = jax.lax.dot(h, v_ref[...])
    e = jnp.exp(beta)
    wgt = (x * e).astype(jnp.bfloat16)
    e_bf = e.astype(jnp.bfloat16)

    ids = bi_ref[0]
    base = (jnp.min(ids) // 8) * 8
    local = ids - base
    nwin = jnp.max(local) // W + 1

    row = jax.lax.broadcasted_iota(jnp.int32, (W, C), 0)

    def window(k):
        oht = (row + k * W == local).astype(jnp.bfloat16)
        win_num = jax.lax.dot(oht, wgt,
                              preferred_element_type=jnp.float32)
        win_den = jax.lax.dot(oht, e_bf,
                              preferred_element_type=jnp.float32)
        b = base + k * W
        acc_ref[pl.ds(b, W), :] += win_num
        den_ref[pl.ds(b, W), :] += win_den

    window(0)

    @pl.when(nwin > 1)
    def _extra():
        jax.lax.fori_loop(1, nwin, lambda k, c: (window(k), c)[1], 0)

    @pl.when(c == G - 1)
    def _fin():
        num = acc_ref[pl.ds(0, S), :]
        den = den_ref[pl.ds(0, S), :]
        out_ref[...] = jnp.where(den > 0, num / den, 0.0)


def kernel(embed, batch_index, W_a, V_a):
    bi3 = batch_index.reshape(G, 1, C)
    return pl.pallas_call(
        _attn_kernel,
        grid=(G,),
        in_specs=[
            pl.BlockSpec((C, D), lambda c: (c, 0)),
            pl.BlockSpec((1, 1, C), lambda c: (c, 0, 0)),
            pl.BlockSpec((D, H), lambda c: (0, 0)),
            pl.BlockSpec((H, 1), lambda c: (0, 0)),
        ],
        out_specs=pl.BlockSpec((S, D), lambda c: (0, 0)),
        out_shape=jax.ShapeDtypeStruct((S, D), jnp.float32),
        scratch_shapes=[
            pltpu.VMEM((S + W, D), jnp.float32),
            pltpu.VMEM((S + W, 1), jnp.float32),
        ],
    )(embed, bi3, W_a, V_a)
